# baseline (speedup 1.0000x reference)
import functools

import jax
import jax.numpy as jnp
from jax import lax
from jax.experimental import pallas as pl
from jax.experimental.pallas import tpu as pltpu
from jax.experimental.pallas import tpu_sc as plsc

N = 10000
BAG = 16
D = 512
E = 160000
TBL = 8193

NSL = 4
SL = 128
NC, NS = 2, 16
NR = 10112
STR = NR // NS
DUMP = N
T = 128
TG = 4
NG = 20
NGD = NG // NC
EW = NG * TG * T

ROW_T = 400

_sc_mesh = plsc.VectorSubcoreMesh(core_axis_name="c", subcore_axis_name="s")


NBUF = 3


def _segsum_body(tbl_hbm, src2_hbm, dst2_hbm, zrow_hbm, out_hbm,
                 src_v, dst_v, bufs, acc_sh, gsems, ssems):
    c = lax.axis_index("c")
    s = lax.axis_index("s")
    for p in range(NSL // NC):
        sidx = c * (NSL // NC) + p
        pltpu.sync_copy(zrow_hbm, acc_sh.at[pl.ds(s * STR, STR)])
        plsc.subcore_barrier()

        @pl.loop(0, NG)
        def _group(g):
            pltpu.sync_copy(src2_hbm.at[sidx, s, g], src_v)
            pltpu.sync_copy(dst2_hbm.at[s, g], dst_v)
            gcp = [None] * TG
            scp = [None] * TG
            for j in range(min(2, TG)):
                gcp[j] = pltpu.async_copy(tbl_hbm.at[src_v.at[j]],
                                          bufs[j % NBUF], gsems[j % NBUF])
            for j in range(TG):
                gcp[j].wait()
                scp[j] = pltpu.async_copy(bufs[j % NBUF],
                                          acc_sh.at[dst_v.at[j]],
                                          ssems[j % NBUF], add=True)
                nj = j + 2
                if nj < TG:
                    if nj - NBUF >= 0:
                        scp[nj - NBUF].wait()
                    gcp[nj] = pltpu.async_copy(tbl_hbm.at[src_v.at[nj]],
                                               bufs[nj % NBUF],
                                               gsems[nj % NBUF])
            for j in range(max(0, TG - NBUF), TG):
                scp[j].wait()

        plsc.subcore_barrier()
        pltpu.sync_copy(acc_sh.at[pl.ds(s * STR, STR)],
                        out_hbm.at[sidx, pl.ds(s * STR, STR)])
        plsc.subcore_barrier()


@functools.partial(
    pl.kernel,
    out_type=jax.ShapeDtypeStruct((NSL, NR, SL), jnp.float32),
    mesh=_sc_mesh,
    scratch_types=[
        pltpu.VMEM((TG, T), jnp.int32),
        pltpu.VMEM((TG, T), jnp.int32),
        pltpu.VMEM((T, SL), jnp.float32),
        pltpu.VMEM((T, SL), jnp.float32),
        pltpu.VMEM((T, SL), jnp.float32),
        pltpu.VMEM_SHARED((NR, SL), jnp.float32),
        pltpu.SemaphoreType.DMA,
        pltpu.SemaphoreType.DMA,
        pltpu.SemaphoreType.DMA,
        pltpu.SemaphoreType.DMA,
        pltpu.SemaphoreType.DMA,
        pltpu.SemaphoreType.DMA,
    ],
)
def _sc_segsum(tbl_hbm, src2_hbm, dst2_hbm, zrow_hbm, out_hbm,
               src_v, dst_v, buf_a, buf_b, buf_c, acc_sh,
               gsem_a, gsem_b, gsem_c, ssem_a, ssem_b, ssem_c):
    _segsum_body(tbl_hbm, src2_hbm, dst2_hbm, zrow_hbm, out_hbm,
                 src_v, dst_v, (buf_a, buf_b, buf_c), acc_sh,
                 (gsem_a, gsem_b, gsem_c), (ssem_a, ssem_b, ssem_c))


@functools.partial(
    pl.kernel,
    out_type=jax.ShapeDtypeStruct((NC, NR, SL), jnp.float32),
    mesh=_sc_mesh,
    scratch_types=[
        pltpu.VMEM((TG, T), jnp.int32),
        pltpu.VMEM((T, SL), jnp.float32),
        pltpu.VMEM_SHARED((NR, SL), jnp.float32),
    ],
)
def _sc_deg(gdst2_hbm, zrow_hbm, ones_hbm, degp_hbm, gdst_v, ones_v, hist_sh):
    c = lax.axis_index("c")
    s = lax.axis_index("s")
    pltpu.sync_copy(ones_hbm, ones_v)
    pltpu.sync_copy(zrow_hbm, hist_sh.at[pl.ds(s * STR, STR)])
    plsc.subcore_barrier()

    @pl.loop(0, NGD)
    def _dgroup(g):
        pltpu.sync_copy(gdst2_hbm.at[s, c * NGD + g], gdst_v)

        @pl.loop(0, TG)
        def _deg(j):
            pltpu.sync_copy(ones_v, hist_sh.at[gdst_v.at[j]], add=True)

    plsc.subcore_barrier()
    pltpu.sync_copy(hist_sh.at[pl.ds(s * STR, STR)],
                    degp_hbm.at[c, pl.ds(s * STR, STR)])


def _dinv_col(dp):
    deg = dp[0, :, 0] + dp[1, :, 0] + 1.0
    return lax.rsqrt(deg)[:, None]


def _hs1_body(h0_ref, dp_ref, o_ref):
    scale = _dinv_col(dp_ref[...]) * (1.0 / BAG)
    for sl in range(NSL):
        o_ref[sl] = h0_ref[sl] * scale


def _hs1_kernel(h0_4, degp):
    return pl.pallas_call(
        _hs1_body,
        grid=(N // ROW_T,),
        in_specs=[
            pl.BlockSpec((NSL, ROW_T, SL), lambda i: (0, i, 0)),
            pl.BlockSpec((NC, ROW_T, SL), lambda i: (0, i, 0)),
        ],
        out_specs=pl.BlockSpec((NSL, ROW_T, SL), lambda i: (0, i, 0)),
        out_shape=jax.ShapeDtypeStruct((NSL, N, SL), jnp.float32),
    )(h0_4, degp)


def _dense_body(hs_ref, s_ref, dp_ref, w_ref, b_ref, o_ref, *, relu, post):
    dinv = _dinv_col(dp_ref[...])
    y = jnp.broadcast_to(b_ref[...], (ROW_T, D))
    for sl in range(NSL):
        xs = (hs_ref[sl] + s_ref[sl]) * dinv
        y = y + jnp.dot(xs, w_ref[pl.ds(sl * SL, SL), :],
                        preferred_element_type=jnp.float32)
    if relu:
        y = jnp.maximum(y, 0.0)
    if post:
        y = y * dinv
        for sl in range(NSL):
            o_ref[sl] = y[:, sl * SL:(sl + 1) * SL]
    else:
        o_ref[...] = y


def _dense_layer(hs_4, S_4, degp, W, b, relu, post):
    if post:
        out_spec = pl.BlockSpec((NSL, ROW_T, SL), lambda i: (0, i, 0))
        out_shape = jax.ShapeDtypeStruct((NSL, N, SL), jnp.float32)
    else:
        out_spec = pl.BlockSpec((ROW_T, D), lambda i: (i, 0))
        out_shape = jax.ShapeDtypeStruct((N, D), jnp.float32)
    return pl.pallas_call(
        functools.partial(_dense_body, relu=relu, post=post),
        grid=(N // ROW_T,),
        in_specs=[
            pl.BlockSpec((NSL, ROW_T, SL), lambda i: (0, i, 0)),
            pl.BlockSpec((NSL, ROW_T, SL), lambda i: (0, i, 0)),
            pl.BlockSpec((NC, ROW_T, SL), lambda i: (0, i, 0)),
            pl.BlockSpec((D, D), lambda i: (0, 0)),
            pl.BlockSpec((1, D), lambda i: (0, 0)),
        ],
        out_specs=out_spec,
        out_shape=out_shape,
    )(hs_4, S_4, degp, W, b.reshape(1, D))


def _stage_edges(idx, pad_val):
    per = E // NS
    idx = idx.astype(jnp.int32).reshape(NS, per)
    idx = jnp.pad(idx, ((0, 0), (0, EW - per)), constant_values=pad_val)
    return idx.reshape(NS, NG, TG, T)


def _slice_offsets(idx2, stride):
    off = (jnp.arange(NSL, dtype=jnp.int32) * stride).reshape(
        NSL, 1, 1, 1, 1)
    return idx2[None] + off


def kernel(x, edge_index, batch, table, W1, b1, W2, b2):
    src, dst = edge_index[0], edge_index[1]

    tblflat = (table.reshape(TBL, NSL, SL).transpose(1, 0, 2)
               .reshape(NSL * TBL, SL))
    src2p = _slice_offsets(_stage_edges(src, 0), N)
    dst2 = _stage_edges(dst, DUMP)
    bsrc2p = _slice_offsets(_stage_edges(x.reshape(-1), 0), TBL)
    bdst2 = _stage_edges(
        jnp.repeat(jnp.arange(N, dtype=jnp.int32), BAG), DUMP)
    zrow = jnp.zeros((STR, SL), jnp.float32)
    onesrow = jnp.ones((T, SL), jnp.float32)

    degp = _sc_deg(dst2, zrow, onesrow)
    h0_4 = _sc_segsum(tblflat, bsrc2p, bdst2, zrow)
    hs1_4 = _hs1_kernel(h0_4, degp)
    S1_4 = _sc_segsum(hs1_4.reshape(NSL * N, SL), src2p, dst2, zrow)
    hs2_4 = _dense_layer(hs1_4, S1_4, degp, W1, b1, relu=True, post=True)
    S2_4 = _sc_segsum(hs2_4.reshape(NSL * N, SL), src2p, dst2, zrow)
    return _dense_layer(hs2_4, S2_4, degp, W2, b2, relu=False, post=False)


# speedup vs baseline: 1.0445x; 1.0445x over previous
import functools

import jax
import jax.numpy as jnp
from jax import lax
from jax.experimental import pallas as pl
from jax.experimental.pallas import tpu as pltpu
from jax.experimental.pallas import tpu_sc as plsc

N = 10000
BAG = 16
D = 512
E = 160000
TBL = 8193

NSL = 4
SL = 128
NC, NS = 2, 16
NR = 10112
STR = NR // NS
DUMP = N
T = 128
TG = 8
NG = 10
NGD = NG // NC
EW = NG * TG * T

ROW_T = 400

_sc_mesh = plsc.VectorSubcoreMesh(core_axis_name="c", subcore_axis_name="s")


def _segsum_body(tbl_hbm, src2_hbm, dst2_hbm, zrow_hbm, out_hbm,
                 src_v, dst_v, bufs, acc_sh, gsems):
    c = lax.axis_index("c")
    s = lax.axis_index("s")
    for p in range(NSL // NC):
        sidx = c * (NSL // NC) + p
        pltpu.sync_copy(zrow_hbm, acc_sh.at[pl.ds(s * STR, STR)])
        plsc.subcore_barrier()

        @pl.loop(0, NG)
        def _group(g):
            pltpu.sync_copy(src2_hbm.at[sidx, s, g], src_v)
            pltpu.sync_copy(dst2_hbm.at[s, g], dst_v)
            cp = pltpu.async_copy(tbl_hbm.at[src_v.at[0]], bufs[0], gsems[0])
            for j in range(TG):
                nxt = None
                if j + 1 < TG:
                    nxt = pltpu.async_copy(tbl_hbm.at[src_v.at[j + 1]],
                                           bufs[(j + 1) % 2],
                                           gsems[(j + 1) % 2])
                cp.wait()
                pltpu.sync_copy(bufs[j % 2], acc_sh.at[dst_v.at[j]], add=True)
                cp = nxt

        plsc.subcore_barrier()
        pltpu.sync_copy(acc_sh.at[pl.ds(s * STR, STR)],
                        out_hbm.at[sidx, pl.ds(s * STR, STR)])
        plsc.subcore_barrier()


@functools.partial(
    pl.kernel,
    out_type=jax.ShapeDtypeStruct((NSL, NR, SL), jnp.float32),
    mesh=_sc_mesh,
    scratch_types=[
        pltpu.VMEM((TG, T), jnp.int32),
        pltpu.VMEM((TG, T), jnp.int32),
        pltpu.VMEM((T, SL), jnp.float32),
        pltpu.VMEM((T, SL), jnp.float32),
        pltpu.VMEM_SHARED((NR, SL), jnp.float32),
        pltpu.SemaphoreType.DMA,
        pltpu.SemaphoreType.DMA,
    ],
)
def _sc_segsum(tbl_hbm, src2_hbm, dst2_hbm, zrow_hbm, out_hbm,
               src_v, dst_v, buf_a, buf_b, acc_sh, gsem_a, gsem_b):
    _segsum_body(tbl_hbm, src2_hbm, dst2_hbm, zrow_hbm, out_hbm,
                 src_v, dst_v, (buf_a, buf_b), acc_sh, (gsem_a, gsem_b))


@functools.partial(
    pl.kernel,
    out_type=jax.ShapeDtypeStruct((NC, NR, SL), jnp.float32),
    mesh=_sc_mesh,
    scratch_types=[
        pltpu.VMEM((TG, T), jnp.int32),
        pltpu.VMEM((T, SL), jnp.float32),
        pltpu.VMEM_SHARED((NR, SL), jnp.float32),
    ],
)
def _sc_deg(gdst2_hbm, zrow_hbm, ones_hbm, degp_hbm, gdst_v, ones_v, hist_sh):
    c = lax.axis_index("c")
    s = lax.axis_index("s")
    pltpu.sync_copy(ones_hbm, ones_v)
    pltpu.sync_copy(zrow_hbm, hist_sh.at[pl.ds(s * STR, STR)])
    plsc.subcore_barrier()

    @pl.loop(0, NGD)
    def _dgroup(g):
        pltpu.sync_copy(gdst2_hbm.at[s, c * NGD + g], gdst_v)

        @pl.loop(0, TG)
        def _deg(j):
            pltpu.sync_copy(ones_v, hist_sh.at[gdst_v.at[j]], add=True)

    plsc.subcore_barrier()
    pltpu.sync_copy(hist_sh.at[pl.ds(s * STR, STR)],
                    degp_hbm.at[c, pl.ds(s * STR, STR)])


def _dinv_col(dp):
    deg = dp[0, :, 0] + dp[1, :, 0] + 1.0
    return lax.rsqrt(deg)[:, None]


def _hs1_body(h0_ref, dp_ref, o_ref):
    scale = _dinv_col(dp_ref[...]) * (1.0 / BAG)
    for sl in range(NSL):
        o_ref[sl] = h0_ref[sl] * scale


def _hs1_kernel(h0_4, degp):
    return pl.pallas_call(
        _hs1_body,
        grid=(N // ROW_T,),
        in_specs=[
            pl.BlockSpec((NSL, ROW_T, SL), lambda i: (0, i, 0)),
            pl.BlockSpec((NC, ROW_T, SL), lambda i: (0, i, 0)),
        ],
        out_specs=pl.BlockSpec((NSL, ROW_T, SL), lambda i: (0, i, 0)),
        out_shape=jax.ShapeDtypeStruct((NSL, N, SL), jnp.float32),
    )(h0_4, degp)


def _dense_body(hs_ref, s_ref, dp_ref, w_ref, b_ref, o_ref, *, relu, post):
    dinv = _dinv_col(dp_ref[...])
    y = jnp.broadcast_to(b_ref[...], (ROW_T, D))
    for sl in range(NSL):
        xs = (hs_ref[sl] + s_ref[sl]) * dinv
        y = y + jnp.dot(xs, w_ref[pl.ds(sl * SL, SL), :],
                        preferred_element_type=jnp.float32)
    if relu:
        y = jnp.maximum(y, 0.0)
    if post:
        y = y * dinv
        for sl in range(NSL):
            o_ref[sl] = y[:, sl * SL:(sl + 1) * SL]
    else:
        o_ref[...] = y


def _dense_layer(hs_4, S_4, degp, W, b, relu, post):
    if post:
        out_spec = pl.BlockSpec((NSL, ROW_T, SL), lambda i: (0, i, 0))
        out_shape = jax.ShapeDtypeStruct((NSL, N, SL), jnp.float32)
    else:
        out_spec = pl.BlockSpec((ROW_T, D), lambda i: (i, 0))
        out_shape = jax.ShapeDtypeStruct((N, D), jnp.float32)
    return pl.pallas_call(
        functools.partial(_dense_body, relu=relu, post=post),
        grid=(N // ROW_T,),
        in_specs=[
            pl.BlockSpec((NSL, ROW_T, SL), lambda i: (0, i, 0)),
            pl.BlockSpec((NSL, ROW_T, SL), lambda i: (0, i, 0)),
            pl.BlockSpec((NC, ROW_T, SL), lambda i: (0, i, 0)),
            pl.BlockSpec((D, D), lambda i: (0, 0)),
            pl.BlockSpec((1, D), lambda i: (0, 0)),
        ],
        out_specs=out_spec,
        out_shape=out_shape,
    )(hs_4, S_4, degp, W, b.reshape(1, D))


def _stage_edges(idx, pad_val):
    per = E // NS
    idx = idx.astype(jnp.int32).reshape(NS, per)
    idx = jnp.pad(idx, ((0, 0), (0, EW - per)), constant_values=pad_val)
    return idx.reshape(NS, NG, TG, T)


def _slice_offsets(idx2, stride):
    off = (jnp.arange(NSL, dtype=jnp.int32) * stride).reshape(
        NSL, 1, 1, 1, 1)
    return idx2[None] + off


def kernel(x, edge_index, batch, table, W1, b1, W2, b2):
    src, dst = edge_index[0], edge_index[1]

    tblflat = (table.reshape(TBL, NSL, SL).transpose(1, 0, 2)
               .reshape(NSL * TBL, SL))
    src2p = _slice_offsets(_stage_edges(src, 0), N)
    dst2 = _stage_edges(dst, DUMP)
    bsrc2p = _slice_offsets(_stage_edges(x.reshape(-1), 0), TBL)
    bdst2 = _stage_edges(
        jnp.repeat(jnp.arange(N, dtype=jnp.int32), BAG), DUMP)
    zrow = jnp.zeros((STR, SL), jnp.float32)
    onesrow = jnp.ones((T, SL), jnp.float32)

    degp = _sc_deg(dst2, zrow, onesrow)
    h0_4 = _sc_segsum(tblflat, bsrc2p, bdst2, zrow)
    hs1_4 = _hs1_kernel(h0_4, degp)
    S1_4 = _sc_segsum(hs1_4.reshape(NSL * N, SL), src2p, dst2, zrow)
    hs2_4 = _dense_layer(hs1_4, S1_4, degp, W1, b1, relu=True, post=True)
    S2_4 = _sc_segsum(hs2_4.reshape(NSL * N, SL), src2p, dst2, zrow)
    return _dense_layer(hs2_4, S2_4, degp, W2, b2, relu=False, post=False)


# speedup vs baseline: 1.0811x; 1.0350x over previous
import functools

import jax
import jax.numpy as jnp
from jax import lax
from jax.experimental import pallas as pl
from jax.experimental.pallas import tpu as pltpu
from jax.experimental.pallas import tpu_sc as plsc

N = 10000
BAG = 16
D = 512
E = 160000
TBL = 8193

NSL = 4
SL = 128
NC, NS = 2, 16
NR = 10112
STR = NR // NS
DUMP = N
T = 128
TG = 8
NG = 10
NGD = NG // NC
EW = NG * TG * T

ROW_T = 400

_sc_mesh = plsc.VectorSubcoreMesh(core_axis_name="c", subcore_axis_name="s")


def _segsum_body(tbl_hbm, comb_hbm, zrow_hbm, out_hbm,
                 idxs, bufs, acc_sh, gsems, isems):
    c = lax.axis_index("c")
    s = lax.axis_index("s")
    for p in range(NSL // NC):
        sidx = c * (NSL // NC) + p
        pltpu.sync_copy(zrow_hbm, acc_sh.at[pl.ds(s * STR, STR)])
        pltpu.async_copy(comb_hbm.at[sidx, s, 0], idxs[0], isems[0])
        plsc.subcore_barrier()

        @pl.loop(0, NG, step=2)
        def _group(g):
            for b in range(2):
                gg = g + b

                @pl.when(gg + 1 < NG)
                def _prefetch():
                    pltpu.async_copy(comb_hbm.at[sidx, s, gg + 1],
                                     idxs[1 - b], isems[1 - b])

                pltpu.make_async_copy(comb_hbm.at[sidx, s, 0], idxs[b],
                                      isems[b]).wait()
                idx_v = idxs[b]
                cp = pltpu.async_copy(tbl_hbm.at[idx_v.at[0]], bufs[0],
                                      gsems[0])
                for j in range(TG):
                    nxt = None
                    if j + 1 < TG:
                        nxt = pltpu.async_copy(tbl_hbm.at[idx_v.at[j + 1]],
                                               bufs[(j + 1) % 2],
                                               gsems[(j + 1) % 2])
                    cp.wait()
                    pltpu.sync_copy(bufs[j % 2], acc_sh.at[idx_v.at[TG + j]],
                                    add=True)
                    cp = nxt

        plsc.subcore_barrier()
        pltpu.sync_copy(acc_sh.at[pl.ds(s * STR, STR)],
                        out_hbm.at[sidx, pl.ds(s * STR, STR)])
        plsc.subcore_barrier()


@functools.partial(
    pl.kernel,
    out_type=jax.ShapeDtypeStruct((NSL, NR, SL), jnp.float32),
    mesh=_sc_mesh,
    scratch_types=[
        pltpu.VMEM((2 * TG, T), jnp.int32),
        pltpu.VMEM((2 * TG, T), jnp.int32),
        pltpu.VMEM((T, SL), jnp.float32),
        pltpu.VMEM((T, SL), jnp.float32),
        pltpu.VMEM_SHARED((NR, SL), jnp.float32),
        pltpu.SemaphoreType.DMA,
        pltpu.SemaphoreType.DMA,
        pltpu.SemaphoreType.DMA,
        pltpu.SemaphoreType.DMA,
    ],
)
def _sc_segsum(tbl_hbm, comb_hbm, zrow_hbm, out_hbm,
               idx_a, idx_b, buf_a, buf_b, acc_sh,
               gsem_a, gsem_b, isem_a, isem_b):
    _segsum_body(tbl_hbm, comb_hbm, zrow_hbm, out_hbm,
                 (idx_a, idx_b), (buf_a, buf_b), acc_sh,
                 (gsem_a, gsem_b), (isem_a, isem_b))


@functools.partial(
    pl.kernel,
    out_type=jax.ShapeDtypeStruct((NC, NR, SL), jnp.float32),
    mesh=_sc_mesh,
    scratch_types=[
        pltpu.VMEM((TG, T), jnp.int32),
        pltpu.VMEM((T, SL), jnp.float32),
        pltpu.VMEM_SHARED((NR, SL), jnp.float32),
    ],
)
def _sc_deg(gdst2_hbm, zrow_hbm, ones_hbm, degp_hbm, gdst_v, ones_v, hist_sh):
    c = lax.axis_index("c")
    s = lax.axis_index("s")
    pltpu.sync_copy(ones_hbm, ones_v)
    pltpu.sync_copy(zrow_hbm, hist_sh.at[pl.ds(s * STR, STR)])
    plsc.subcore_barrier()

    @pl.loop(0, NGD)
    def _dgroup(g):
        pltpu.sync_copy(gdst2_hbm.at[s, c * NGD + g], gdst_v)

        @pl.loop(0, TG)
        def _deg(j):
            pltpu.sync_copy(ones_v, hist_sh.at[gdst_v.at[j]], add=True)

    plsc.subcore_barrier()
    pltpu.sync_copy(hist_sh.at[pl.ds(s * STR, STR)],
                    degp_hbm.at[c, pl.ds(s * STR, STR)])


def _dinv_col(dp):
    deg = dp[0, :, 0] + dp[1, :, 0] + 1.0
    return lax.rsqrt(deg)[:, None]


def _hs1_body(h0_ref, dp_ref, o_ref):
    scale = _dinv_col(dp_ref[...]) * (1.0 / BAG)
    for sl in range(NSL):
        o_ref[sl] = h0_ref[sl] * scale


def _hs1_kernel(h0_4, degp):
    return pl.pallas_call(
        _hs1_body,
        grid=(N // ROW_T,),
        in_specs=[
            pl.BlockSpec((NSL, ROW_T, SL), lambda i: (0, i, 0)),
            pl.BlockSpec((NC, ROW_T, SL), lambda i: (0, i, 0)),
        ],
        out_specs=pl.BlockSpec((NSL, ROW_T, SL), lambda i: (0, i, 0)),
        out_shape=jax.ShapeDtypeStruct((NSL, N, SL), jnp.float32),
    )(h0_4, degp)


def _dense_body(hs_ref, s_ref, dp_ref, w_ref, b_ref, o_ref, *, relu, post):
    dinv = _dinv_col(dp_ref[...])
    y = jnp.broadcast_to(b_ref[...], (ROW_T, D))
    for sl in range(NSL):
        xs = (hs_ref[sl] + s_ref[sl]) * dinv
        y = y + jnp.dot(xs, w_ref[pl.ds(sl * SL, SL), :],
                        preferred_element_type=jnp.float32)
    if relu:
        y = jnp.maximum(y, 0.0)
    if post:
        y = y * dinv
        for sl in range(NSL):
            o_ref[sl] = y[:, sl * SL:(sl + 1) * SL]
    else:
        o_ref[...] = y


def _dense_layer(hs_4, S_4, degp, W, b, relu, post):
    if post:
        out_spec = pl.BlockSpec((NSL, ROW_T, SL), lambda i: (0, i, 0))
        out_shape = jax.ShapeDtypeStruct((NSL, N, SL), jnp.float32)
    else:
        out_spec = pl.BlockSpec((ROW_T, D), lambda i: (i, 0))
        out_shape = jax.ShapeDtypeStruct((N, D), jnp.float32)
    return pl.pallas_call(
        functools.partial(_dense_body, relu=relu, post=post),
        grid=(N // ROW_T,),
        in_specs=[
            pl.BlockSpec((NSL, ROW_T, SL), lambda i: (0, i, 0)),
            pl.BlockSpec((NSL, ROW_T, SL), lambda i: (0, i, 0)),
            pl.BlockSpec((NC, ROW_T, SL), lambda i: (0, i, 0)),
            pl.BlockSpec((D, D), lambda i: (0, 0)),
            pl.BlockSpec((1, D), lambda i: (0, 0)),
        ],
        out_specs=out_spec,
        out_shape=out_shape,
    )(hs_4, S_4, degp, W, b.reshape(1, D))


def _stage_edges(idx, pad_val):
    per = E // NS
    idx = idx.astype(jnp.int32).reshape(NS, per)
    idx = jnp.pad(idx, ((0, 0), (0, EW - per)), constant_values=pad_val)
    return idx.reshape(NS, NG, TG, T)


def _slice_offsets(idx2, stride):
    off = (jnp.arange(NSL, dtype=jnp.int32) * stride).reshape(
        NSL, 1, 1, 1, 1)
    return idx2[None] + off


def _pack(src2p, dst2):
    dst_b = jnp.broadcast_to(dst2[None], (NSL,) + dst2.shape)
    return jnp.concatenate([src2p, dst_b], axis=3)


def kernel(x, edge_index, batch, table, W1, b1, W2, b2):
    src, dst = edge_index[0], edge_index[1]

    tblflat = (table.reshape(TBL, NSL, SL).transpose(1, 0, 2)
               .reshape(NSL * TBL, SL))
    src2p = _slice_offsets(_stage_edges(src, 0), N)
    dst2 = _stage_edges(dst, DUMP)
    bsrc2p = _slice_offsets(_stage_edges(x.reshape(-1), 0), TBL)
    bdst2 = _stage_edges(
        jnp.repeat(jnp.arange(N, dtype=jnp.int32), BAG), DUMP)
    zrow = jnp.zeros((STR, SL), jnp.float32)
    onesrow = jnp.ones((T, SL), jnp.float32)

    comb_bag = _pack(bsrc2p, bdst2)
    comb_g = _pack(src2p, dst2)

    degp = _sc_deg(dst2, zrow, onesrow)
    h0_4 = _sc_segsum(tblflat, comb_bag, zrow)
    hs1_4 = _hs1_kernel(h0_4, degp)
    S1_4 = _sc_segsum(hs1_4.reshape(NSL * N, SL), comb_g, zrow)
    hs2_4 = _dense_layer(hs1_4, S1_4, degp, W1, b1, relu=True, post=True)
    S2_4 = _sc_segsum(hs2_4.reshape(NSL * N, SL), comb_g, zrow)
    return _dense_layer(hs2_4, S2_4, degp, W2, b2, relu=False, post=False)
